# baseline (speedup 1.0000x reference)
import functools

import jax
import jax.numpy as jnp
from jax import lax
from jax.experimental import pallas as pl
from jax.experimental.pallas import tpu as pltpu
from jax.experimental.pallas import tpu_sc as plsc

_D = 64
_B = 4096
_H = 200

_NC = 2
_NS = 16
_NW = _NC * _NS
_BB = _B // _NW
_L = 16


def _embed_gather(weight, idx_t):
  mesh = plsc.VectorSubcoreMesh(core_axis_name="c", subcore_axis_name="s")

  @functools.partial(
      pl.kernel,
      mesh=mesh,
      out_type=jax.ShapeDtypeStruct((_H, 8, _NW, 8, 128), jnp.float32),
      compiler_params=pltpu.CompilerParams(
          use_tc_tiling_on_sc=False, needs_layout_passes=False),
      scratch_types=[
          pltpu.VMEM((_H, _BB), jnp.int32),
          pltpu.VMEM((_BB, _D), jnp.float32),
          pltpu.VMEM((_BB, _D), jnp.float32),
          pltpu.VMEM((8, 8, 128), jnp.float32),
          pltpu.VMEM((8, 8, 128), jnp.float32),
          pltpu.SemaphoreType.DMA,
          pltpu.SemaphoreType.DMA,
          pltpu.SemaphoreType.DMA,
          pltpu.SemaphoreType.DMA,
          pltpu.SemaphoreType.DMA,
      ],
  )
  def k(table_hbm, idx_hbm, out_hbm, idx_v, ga, gb, ta, tb,
        ig, sg0, sg1, so0, so1):
    wid = lax.axis_index("s") * _NC + lax.axis_index("c")
    pltpu.async_copy(
        idx_hbm.at[:, pl.ds(wid * _BB, _BB)], idx_v, ig).wait()

    lanes = lax.iota(jnp.int32, _L)

    def fire_gather(h, buf, sem):
      pltpu.async_copy(table_hbm.at[idx_v.at[h]], buf, sem)

    def drain_gather(h, buf, sem):
      pltpu.make_async_copy(table_hbm.at[idx_v.at[h]], buf, sem).wait()

    def transpose(gbuf, tbuf):
      for big in range(8):
        for r in range(8):
          d = jnp.broadcast_to(jnp.int32(big * 8 + r), (_L,))
          for g in range(8):
            rows = lanes + (g * _L)
            vals = plsc.load_gather(gbuf, [rows, d])
            tbuf[big, r, pl.ds(g * _L, _L)] = vals

    def fire_out(h, tbuf, sem):
      pltpu.async_copy(tbuf, out_hbm.at[h, :, wid], sem)

    def drain_out(h, tbuf, sem):
      pltpu.make_async_copy(tbuf, out_hbm.at[h, :, wid], sem).wait()

    fire_gather(0, ga, sg0)

    def body(t, carry):
      a = 2 * t
      b = a + 1

      @pl.when(t > 0)
      def _():
        drain_out(a - 2, ta, so0)

      fire_gather(b, gb, sg1)
      drain_gather(a, ga, sg0)
      transpose(ga, ta)
      fire_out(a, ta, so0)

      @pl.when(t < _H // 2 - 1)
      def _():
        fire_gather(a + 2, ga, sg0)

      @pl.when(t > 0)
      def _():
        drain_out(b - 2, tb, so1)

      drain_gather(b, gb, sg1)
      transpose(gb, tb)
      fire_out(b, tb, so1)
      return carry

    lax.fori_loop(0, _H // 2, body, 0)
    drain_out(_H - 2, ta, so0)
    drain_out(_H - 1, tb, so1)

  return k(weight, idx_t)


def kernel(indices, weight):
  idx_t = indices.T
  out5 = _embed_gather(weight, idx_t)
  out = out5.transpose(2, 4, 0, 1, 3).reshape(_B, _H, _D)
  return out
